# baseline (speedup 1.0000x reference)
import jax
import jax.numpy as jnp
from jax.experimental import pallas as pl
from jax.experimental.pallas import tpu as pltpu

_T = 1024
_OUT = 1000
_N = 192


def _pair_weight(m, stride):
    k, wc, n = m.shape
    core = jnp.pad(m, ((0, 0), (0, stride - wc), (0, 0))).reshape(k * stride, n)
    even = jnp.pad(core, ((0, stride), (0, 0)))
    odd = jnp.pad(core, ((stride, 0), (0, 0)))
    return jnp.concatenate([even, odd], axis=1)


def _pad_csel(cs):
    h = cs.shape[1] // 2
    z = jnp.zeros((cs.shape[0], 128 - h), cs.dtype)
    return jnp.concatenate([cs[:, :h], z, cs[:, h:], z], axis=1)


def _conv_pair_pool(B, w_ref, b_ref, cs_ref):
    acc = jnp.dot(B, w_ref[...], preferred_element_type=jnp.float32)
    o = (acc + b_ref[...]).astype(jnp.bfloat16)
    r = jnp.maximum(o[:, :_N], o[:, _N:])
    cp = jnp.dot(r, cs_ref[...], preferred_element_type=jnp.float32)
    return jnp.maximum(cp[:, :128], cp[:, 128:]).astype(jnp.bfloat16)


def _pair_windows(P, hin, k, pad, hp, t):
    zero = jnp.zeros((t, 128), jnp.bfloat16)

    def row(j):
        return P[j * t:(j + 1) * t] if 0 <= j < hin else zero

    wins = [jnp.concatenate([row(2 * i - pad + d) for d in range(k + 1)],
                            axis=1) for i in range(hp)]
    return jnp.concatenate(wins, axis=0)


def _fwd_kernel(x_ref,
                w1, b1, cs1,
                w2, b2, cs2,
                w3, b3, cs3,
                w4, b4, cs4,
                fw1, fb1, fw2, fb2,
                out_ref):
    t = x_ref.shape[0]
    xb = x_ref[...].astype(jnp.bfloat16)
    z = jnp.zeros((t, 32), jnp.bfloat16)
    xp = jnp.concatenate(
        [z, z] + [xb[:, 0, j, :] for j in range(32)] + [z, z],
        axis=1)

    B1 = jnp.concatenate([xp[:, 64 * i: 64 * i + 192] for i in range(16)],
                         axis=0)
    P1 = _conv_pair_pool(B1, w1, b1, cs1)

    B2 = _pair_windows(P1, 16, 5, 0, 6, t)
    P2 = _conv_pair_pool(B2, w2, b2, cs2)

    B3 = _pair_windows(P2, 6, 3, 1, 3, t)
    P3 = _conv_pair_pool(B3, w3, b3, cs3)

    B4 = _pair_windows(P3, 3, 3, 1, 1, t)
    f = _conv_pair_pool(B4, w4, b4, cs4)

    h = jnp.dot(f, fw1[...], preferred_element_type=jnp.float32) + fb1[...]
    y = jnp.dot(h.astype(jnp.bfloat16), fw2[...],
                preferred_element_type=jnp.float32) + fb2[...]
    out_ref[...] = y[:, :_OUT]


def _const_specs(arrays):
    return [pl.BlockSpec(a.shape, lambda i, _nd=a.ndim: (0,) * _nd)
            for a in arrays]


def kernel(c1_m, c1_b, c1_rsel, c1_csel,
           c2_m, c2_b, c2_rsel, c2_csel,
           c3_m, c3_b, c3_rsel, c3_csel,
           c4_m, c4_b, c4_rsel, c4_csel,
           fc1_w, fc1_b, fc2_w, fc2_b,
           x):
    n = x.shape[0]
    n_pad = ((n + _T - 1) // _T) * _T
    x2 = x
    if n_pad != n:
        x2 = jnp.concatenate(
            [x2, jnp.zeros((n_pad - n,) + x.shape[1:], x.dtype)], axis=0)

    def bias2(b):
        return jnp.concatenate([b, b], axis=1)

    consts = [
        _pair_weight(c1_m, 32), bias2(c1_b), _pad_csel(c1_csel),
        _pair_weight(c2_m, 128), bias2(c2_b), _pad_csel(c2_csel),
        _pair_weight(c3_m, 128), bias2(c3_b), _pad_csel(c3_csel),
        _pair_weight(c4_m, 128), bias2(c4_b), _pad_csel(c4_csel),
        jnp.pad(fc1_w, ((0, 64), (0, 0))), fc1_b,
        fc2_w, fc2_b,
    ]
    weight_bytes = sum(int(a.size) * a.dtype.itemsize for a in consts)

    out = pl.pallas_call(
        _fwd_kernel,
        out_shape=jax.ShapeDtypeStruct((n_pad, _OUT), jnp.float32),
        grid=(n_pad // _T,),
        in_specs=[pl.BlockSpec((_T, 1, 32, 32), lambda i: (i, 0, 0, 0))]
                 + _const_specs(consts),
        out_specs=pl.BlockSpec((_T, _OUT), lambda i: (i, 0)),
        compiler_params=pltpu.CompilerParams(
            dimension_semantics=("parallel",),
            vmem_limit_bytes=64 * 1024 * 1024),
        cost_estimate=pl.CostEstimate(
            flops=7_500_000 * n_pad,
            transcendentals=0,
            bytes_accessed=weight_bytes + n_pad * (32 * 32 * 4 + _OUT * 4)),
    )(x2, *consts)
    return out[:n]


# speedup vs baseline: 2.8275x; 2.8275x over previous
import jax
import jax.numpy as jnp
from jax.experimental import pallas as pl
from jax.experimental.pallas import tpu as pltpu

_T = 1024
_OUT = 1000
_N = 192


def _pair_weight_t(m, stride):
    k, wc, n = m.shape
    core = jnp.pad(m, ((0, 0), (0, stride - wc), (0, 0))).reshape(k * stride, n)
    even = jnp.pad(core, ((0, stride), (0, 0)))
    odd = jnp.pad(core, ((stride, 0), (0, 0)))
    return jnp.concatenate([even, odd], axis=1).T


def _pad_csel_t(cs):
    h = cs.shape[1] // 2
    z = jnp.zeros((cs.shape[0], 128 - h), cs.dtype)
    return jnp.concatenate([cs[:, :h], z, cs[:, h:], z], axis=1).T


def _conv_pair_pool(B, w_ref, b_ref, cs_ref):
    acc = jnp.dot(w_ref[...], B, preferred_element_type=jnp.float32)
    m0 = jnp.maximum(acc[:_N], acc[_N:])
    r = (m0 + b_ref[...]).astype(jnp.bfloat16)
    cp = jnp.dot(cs_ref[...], r, preferred_element_type=jnp.float32)
    return jnp.maximum(cp[:128], cp[128:]).astype(jnp.bfloat16)


def _pair_windows(P, hin, k, pad, hp, t):
    zero = jnp.zeros((128, t), jnp.bfloat16)

    def row(j):
        return P[:, j * t:(j + 1) * t] if 0 <= j < hin else zero

    wins = [jnp.concatenate([row(2 * i - pad + d) for d in range(k + 1)],
                            axis=0) for i in range(hp)]
    return jnp.concatenate(wins, axis=1)


def _fwd_kernel(x_ref,
                w1, b1, cs1,
                w2, b2, cs2,
                w3, b3, cs3,
                w4, b4, cs4,
                fw1, fb1, fw2, fb2,
                out_ref):
    t = x_ref.shape[1]
    xb = x_ref[...].astype(jnp.bfloat16)
    z = jnp.zeros((64, t), jnp.bfloat16)
    xp = jnp.concatenate([z, xb, z], axis=0)

    B1 = jnp.concatenate([xp[64 * i: 64 * i + 192] for i in range(16)],
                         axis=1)
    P1 = _conv_pair_pool(B1, w1, b1, cs1)

    B2 = _pair_windows(P1, 16, 5, 0, 6, t)
    P2 = _conv_pair_pool(B2, w2, b2, cs2)

    B3 = _pair_windows(P2, 6, 3, 1, 3, t)
    P3 = _conv_pair_pool(B3, w3, b3, cs3)

    B4 = _pair_windows(P3, 3, 3, 1, 1, t)
    f = _conv_pair_pool(B4, w4, b4, cs4)

    h = jnp.dot(fw1[...], f, preferred_element_type=jnp.float32) + fb1[...]
    y = jnp.dot(fw2[...], h.astype(jnp.bfloat16),
                preferred_element_type=jnp.float32) + fb2[...]
    out_ref[...] = y[:_OUT]


def _const_specs(arrays):
    return [pl.BlockSpec(a.shape, lambda i, _nd=a.ndim: (0,) * _nd)
            for a in arrays]


def kernel(c1_m, c1_b, c1_rsel, c1_csel,
           c2_m, c2_b, c2_rsel, c2_csel,
           c3_m, c3_b, c3_rsel, c3_csel,
           c4_m, c4_b, c4_rsel, c4_csel,
           fc1_w, fc1_b, fc2_w, fc2_b,
           x):
    n = x.shape[0]
    xt = x.reshape(n, 32 * 32).T
    n_pad = ((n + _T - 1) // _T) * _T
    if n_pad != n:
        xt = jnp.concatenate(
            [xt, jnp.zeros((32 * 32, n_pad - n), xt.dtype)], axis=1)

    def bias_t(b):
        return b.T

    consts = [
        _pair_weight_t(c1_m, 32), bias_t(c1_b), _pad_csel_t(c1_csel),
        _pair_weight_t(c2_m, 128), bias_t(c2_b), _pad_csel_t(c2_csel),
        _pair_weight_t(c3_m, 128), bias_t(c3_b), _pad_csel_t(c3_csel),
        _pair_weight_t(c4_m, 128), bias_t(c4_b), _pad_csel_t(c4_csel),
        jnp.pad(fc1_w, ((0, 64), (0, 0))).T, fc1_b.T,
        fc2_w.T, fc2_b.T,
    ]
    weight_bytes = sum(int(a.size) * a.dtype.itemsize for a in consts)

    out = pl.pallas_call(
        _fwd_kernel,
        out_shape=jax.ShapeDtypeStruct((_OUT, n_pad), jnp.float32),
        grid=(n_pad // _T,),
        in_specs=[pl.BlockSpec((32 * 32, _T), lambda i: (0, i))]
                 + _const_specs(consts),
        out_specs=pl.BlockSpec((_OUT, _T), lambda i: (0, i)),
        compiler_params=pltpu.CompilerParams(
            dimension_semantics=("parallel",),
            vmem_limit_bytes=64 * 1024 * 1024),
        cost_estimate=pl.CostEstimate(
            flops=7_500_000 * n_pad,
            transcendentals=0,
            bytes_accessed=weight_bytes + n_pad * (32 * 32 * 4 + _OUT * 4)),
    )(xt, *consts)
    return out.T[:n]


# speedup vs baseline: 3.4371x; 1.2156x over previous
import jax
import jax.numpy as jnp
from jax.experimental import pallas as pl
from jax.experimental.pallas import tpu as pltpu

_T = 1024
_OUT = 1000
_N = 192


def _dott(w_ref, B):
    return jax.lax.dot_general(w_ref[...], B, (((0,), (0,)), ((), ())),
                               preferred_element_type=jnp.float32)


def _conv_pool(B, m_ref, b_ref, cs_ref, hp, t):
    acc = _dott(m_ref, B)
    m0 = jnp.maximum(acc[:, :hp * t], acc[:, hp * t:])
    r = (m0 + b_ref[...]).astype(jnp.bfloat16)
    cp = _dott(cs_ref, r)
    h2 = cp.shape[0] // 2
    return jnp.maximum(cp[:h2], cp[h2:]).astype(jnp.bfloat16)


def _windows(P, hin, k, pad, hp, t, wc):
    zero = jnp.zeros((wc, t), jnp.bfloat16)

    def row(j):
        return P[:, j * t:(j + 1) * t] if 0 <= j < hin else zero

    def win(i):
        return jnp.concatenate([row(i - pad + d) for d in range(k)], axis=0)

    order = [2 * i for i in range(hp)] + [2 * i + 1 for i in range(hp)]
    return jnp.concatenate([win(i) for i in order], axis=1)


def _fwd_kernel(x_ref,
                m1, b1, cs1,
                m2, b2, cs2,
                m3, b3, cs3,
                m4, b4, cs4,
                fw1, fb1, fw2, fb2,
                out_ref):
    t = x_ref.shape[1]
    xb = x_ref[...].astype(jnp.bfloat16)
    z = jnp.zeros((64, t), jnp.bfloat16)
    xp = jnp.concatenate([z, xb, z], axis=0)

    B1 = jnp.concatenate(
        [xp[64 * j: 64 * j + 160] for j in range(16)]
        + [xp[64 * j + 32: 64 * j + 192] for j in range(16)],
        axis=1)
    P1 = _conv_pool(B1, m1, b1, cs1, 16, t)

    B2 = _windows(P1, 16, 5, 0, 6, t, 96)
    P2 = _conv_pool(B2, m2, b2, cs2, 6, t)

    B3 = _windows(P2, 6, 3, 1, 3, t, 96)
    P3 = _conv_pool(B3, m3, b3, cs3, 3, t)

    B4 = _windows(P3, 3, 3, 1, 1, t, 96)
    f = _conv_pool(B4, m4, b4, cs4, 1, t)

    h = _dott(fw1, f) + fb1[...]
    y = _dott(fw2, h.astype(jnp.bfloat16)) + fb2[...]
    out_ref[...] = y[:_OUT]


def _const_specs(arrays):
    return [pl.BlockSpec(a.shape, lambda i, _nd=a.ndim: (0,) * _nd)
            for a in arrays]


def kernel(c1_m, c1_b, c1_rsel, c1_csel,
           c2_m, c2_b, c2_rsel, c2_csel,
           c3_m, c3_b, c3_rsel, c3_csel,
           c4_m, c4_b, c4_rsel, c4_csel,
           fc1_w, fc1_b, fc2_w, fc2_b,
           x):
    n = x.shape[0]
    xt = x.reshape(n, 32 * 32).T
    n_pad = ((n + _T - 1) // _T) * _T
    if n_pad != n:
        xt = jnp.concatenate(
            [xt, jnp.zeros((32 * 32, n_pad - n), xt.dtype)], axis=1)

    consts = [
        c1_m.reshape(160, _N), c1_b.T, c1_csel,
        c2_m.reshape(480, _N), c2_b.T, c2_csel,
        c3_m.reshape(288, _N), c3_b.T, c3_csel,
        c4_m.reshape(288, _N), c4_b.T, c4_csel,
        fc1_w, fc1_b.T, fc2_w, fc2_b.T,
    ]
    weight_bytes = sum(int(a.size) * a.dtype.itemsize for a in consts)

    out = pl.pallas_call(
        _fwd_kernel,
        out_shape=jax.ShapeDtypeStruct((_OUT, n_pad), jnp.float32),
        grid=(n_pad // _T,),
        in_specs=[pl.BlockSpec((32 * 32, _T), lambda i: (0, i))]
                 + _const_specs(consts),
        out_specs=pl.BlockSpec((_OUT, _T), lambda i: (0, i)),
        compiler_params=pltpu.CompilerParams(
            dimension_semantics=("parallel",),
            vmem_limit_bytes=64 * 1024 * 1024),
        cost_estimate=pl.CostEstimate(
            flops=7_500_000 * n_pad,
            transcendentals=0,
            bytes_accessed=weight_bytes + n_pad * (32 * 32 * 4 + _OUT * 4)),
    )(xt, *consts)
    return out.T[:n]


# speedup vs baseline: 3.9065x; 1.1366x over previous
import jax
import jax.numpy as jnp
from jax.experimental import pallas as pl
from jax.experimental.pallas import tpu as pltpu

_T = 1024
_OUT = 1000
_N = 192


def _dott(w_ref, B):
    return jax.lax.dot_general(w_ref[...], B, (((0,), (0,)), ((), ())),
                               preferred_element_type=jnp.float32)


def _pool_perm(m2d, wo, c, keep):
    m3 = m2d.reshape(m2d.shape[0], wo, c)
    ev = m3[:, 0:2 * (keep // (2 * c)):2]
    od = m3[:, 1:2 * (keep // (2 * c)):2]
    return jnp.concatenate([ev, od], axis=1).reshape(m2d.shape[0], keep)


def _conv_pool(B, m_ref, b_ref, hp, t):
    acc = _dott(m_ref, B)
    m0 = jnp.maximum(acc[:, :hp * t], acc[:, hp * t:])
    s = m0.shape[0] // 2
    m1 = jnp.maximum(m0[:s], m0[s:])
    return (m1 + b_ref[...]).astype(jnp.bfloat16)


def _windows(P, hin, k, pad, hp, t, wc):
    zero = jnp.zeros((wc, t), jnp.bfloat16)

    def row(j):
        return P[:, j * t:(j + 1) * t] if 0 <= j < hin else zero

    def win(i):
        return jnp.concatenate([row(i - pad + d) for d in range(k)], axis=0)

    order = [2 * i for i in range(hp)] + [2 * i + 1 for i in range(hp)]
    return jnp.concatenate([win(i) for i in order], axis=1)


def _fwd_kernel(x_ref,
                m1, b1, m2, b2, m3, b3, m4, b4,
                fw1, fb1, fw2, fb2,
                out_ref):
    t = x_ref.shape[1]
    xb = x_ref[...].astype(jnp.bfloat16)
    z = jnp.zeros((64, t), jnp.bfloat16)
    xp = jnp.concatenate([z, xb, z], axis=0)

    B1 = jnp.concatenate(
        [xp[64 * j: 64 * j + 160] for j in range(16)]
        + [xp[64 * j + 32: 64 * j + 192] for j in range(16)],
        axis=1)
    P1 = _conv_pool(B1, m1, b1, 16, t)

    B2 = _windows(P1, 16, 5, 0, 6, t, 96)
    P2 = _conv_pool(B2, m2, b2, 6, t)

    B3 = _windows(P2, 6, 3, 1, 3, t, 96)
    P3 = _conv_pool(B3, m3, b3, 3, t)

    B4 = _windows(P3, 3, 3, 1, 1, t, 96)
    f = _conv_pool(B4, m4, b4, 1, t)

    h = _dott(fw1, f) + fb1[...]
    y = _dott(fw2, h.astype(jnp.bfloat16)) + fb2[...]
    out_ref[...] = y[:_OUT]


def _const_specs(arrays):
    return [pl.BlockSpec(a.shape, lambda i, _nd=a.ndim: (0,) * _nd)
            for a in arrays]


def kernel(c1_m, c1_b, c1_rsel, c1_csel,
           c2_m, c2_b, c2_rsel, c2_csel,
           c3_m, c3_b, c3_rsel, c3_csel,
           c4_m, c4_b, c4_rsel, c4_csel,
           fc1_w, fc1_b, fc2_w, fc2_b,
           x):
    n = x.shape[0]
    xt = x.reshape(n, 32 * 32).T
    n_pad = ((n + _T - 1) // _T) * _T
    if n_pad != n:
        xt = jnp.concatenate(
            [xt, jnp.zeros((32 * 32, n_pad - n), xt.dtype)], axis=1)

    consts = [
        _pool_perm(c1_m.reshape(160, _N), 32, 6, _N), c1_b[:, :96].T,
        _pool_perm(c2_m.reshape(480, _N), 12, 16, _N), c2_b[:, :96].T,
        _pool_perm(c3_m.reshape(288, _N), 6, 32, _N), c3_b[:, :96].T,
        _pool_perm(c4_m.reshape(288, _N), 3, 64, 128), c4_b[:, :64].T,
        fc1_w, fc1_b.T, fc2_w, fc2_b.T,
    ]
    weight_bytes = sum(int(a.size) * a.dtype.itemsize for a in consts)

    out = pl.pallas_call(
        _fwd_kernel,
        out_shape=jax.ShapeDtypeStruct((_OUT, n_pad), jnp.float32),
        grid=(n_pad // _T,),
        in_specs=[pl.BlockSpec((32 * 32, _T), lambda i: (0, i))]
                 + _const_specs(consts),
        out_specs=pl.BlockSpec((_OUT, _T), lambda i: (0, i)),
        compiler_params=pltpu.CompilerParams(
            dimension_semantics=("parallel",),
            vmem_limit_bytes=64 * 1024 * 1024),
        cost_estimate=pl.CostEstimate(
            flops=7_500_000 * n_pad,
            transcendentals=0,
            bytes_accessed=weight_bytes + n_pad * (32 * 32 * 4 + _OUT * 4)),
    )(xt, *consts)
    return out.T[:n]
